# baseline (speedup 1.0000x reference)
import functools

import jax
import jax.numpy as jnp
from jax import lax
from jax.experimental import pallas as pl
from jax.experimental.pallas import tpu as pltpu
from jax.experimental.pallas import tpu_sc as plsc

_N = 10000
_E = 320000
_D = 128
_H = 128

_NW = 32
_NBINS = 10240
_PER_TILE = _NBINS // 16

_CHUNK = 10240
_TAIL = _E - 31 * _CHUNK


@functools.cache
def _make_sc_histogram():
    mesh = plsc.VectorSubcoreMesh(core_axis_name="c", subcore_axis_name="s")
    return functools.partial(
        pl.kernel,
        out_type=(jax.ShapeDtypeStruct((1, _NBINS), jnp.float32),
                  jax.ShapeDtypeStruct((1, _NBINS), jnp.float32)),
        mesh=mesh,
        scratch_types=[
            pltpu.VMEM((2, _CHUNK), jnp.int32),
            pltpu.VMEM((_CHUNK,), jnp.int32),
            pltpu.VMEM((_CHUNK,), jnp.float32),
            pltpu.VMEM((_PER_TILE,), jnp.float32),
            pltpu.VMEM_SHARED((_NBINS,), jnp.float32),
        ],
    )(_sc_histogram_body)


def _sc_histogram_body(edge_hbm, out0_hbm, out1_hbm, idx_v, idx1_v, ones_v,
                       zeros_v, counts_sh):
    cid = lax.axis_index("c")
    sid = lax.axis_index("s")
    wid = sid * 2 + cid

    one16 = jnp.ones((16,), jnp.float32)
    zero16 = jnp.zeros((16,), jnp.float32)
    for i in range(_CHUNK // 16):
        ones_v[pl.ds(i * 16, 16)] = one16
    for i in range(_PER_TILE // 16):
        zeros_v[pl.ds(i * 16, 16)] = zero16

    pltpu.sync_copy(zeros_v, counts_sh.at[pl.ds(sid * _PER_TILE, _PER_TILE)])
    @pl.when(wid < 31)
    def _():
        pltpu.sync_copy(edge_hbm.at[:, pl.ds(wid * _CHUNK, _CHUNK)], idx_v)

    @pl.when(wid == 31)
    def _():
        pltpu.sync_copy(edge_hbm.at[:, pl.ds(31 * _CHUNK, _TAIL)],
                        idx_v.at[:, pl.ds(0, _TAIL)])

    def compact(j, carry):
        idx1_v[pl.ds(j * 16, 16)] = idx_v[0, pl.ds(j * 16, 16)]
        return carry

    lax.fori_loop(0, _CHUNK // 16, compact, 0)
    plsc.subcore_barrier()

    @pl.when(wid < 31)
    def _():
        pltpu.sync_copy(ones_v, counts_sh.at[idx1_v], add=True)

    @pl.when(wid == 31)
    def _():
        pltpu.sync_copy(ones_v.at[pl.ds(0, _TAIL)],
                        counts_sh.at[idx1_v.at[pl.ds(0, _TAIL)]], add=True)

    plsc.subcore_barrier()

    @pl.when(cid == 0)
    def _():
        pltpu.sync_copy(counts_sh.at[pl.ds(sid * _PER_TILE, _PER_TILE)],
                        out0_hbm.at[0, pl.ds(sid * _PER_TILE, _PER_TILE)])

    @pl.when(cid == 1)
    def _():
        pltpu.sync_copy(counts_sh.at[pl.ds(sid * _PER_TILE, _PER_TILE)],
                        out1_hbm.at[0, pl.ds(sid * _PER_TILE, _PER_TILE)])


def _tc_node_head(x_ref, w1_ref, woutt_ref, y_ref):
    hi = jax.lax.Precision.HIGHEST
    xv = x_ref[...].astype(jnp.bfloat16).astype(jnp.float32)
    w1 = w1_ref[...].astype(jnp.bfloat16).astype(jnp.float32)
    w_row = lax.dot_general(woutt_ref[...], w1, (((1,), (1,)), ((), ())),
                            precision=hi,
                            preferred_element_type=jnp.float32)
    y_ref[...] = lax.dot_general(w_row, xv, (((1,), (1,)), ((), ())),
                                 precision=hi,
                                 preferred_element_type=jnp.float32)


def _tc_final(c0_ref, c1_ref, y_ref, b1_ref, wout_ref, bout_ref, o_ref):
    hi = jax.lax.Precision.HIGHEST
    c1 = (c0_ref[...] + c1_ref[...])[:, :_N]
    out = lax.dot_general(c1, y_ref[...], (((1,), (1,)), ((), ())),
                          precision=hi, preferred_element_type=jnp.float32)
    bias = lax.dot_general(b1_ref[...], wout_ref[...],
                           (((1,), (1,)), ((), ())), precision=hi,
                           preferred_element_type=jnp.float32)
    o_ref[...] = out + jnp.float32(_N) * bias + bout_ref[...]


def kernel(x, edge_index, W1, b1, W_out, b_out):
    counts0, counts1 = _make_sc_histogram()(edge_index)

    y = pl.pallas_call(
        _tc_node_head,
        out_shape=jax.ShapeDtypeStruct((1, _N), jnp.float32),
    )(x, W1, W_out.reshape(1, _H))

    out = pl.pallas_call(
        _tc_final,
        out_shape=jax.ShapeDtypeStruct((1, 1), jnp.float32),
    )(counts0, counts1, y, b1.reshape(1, _H), W_out.reshape(1, _H),
      b_out.reshape(1, 1))
    return out


# speedup vs baseline: 1.0257x; 1.0257x over previous
import functools

import jax
import jax.numpy as jnp
from jax import lax
from jax.experimental import pallas as pl
from jax.experimental.pallas import tpu as pltpu
from jax.experimental.pallas import tpu_sc as plsc

_N = 10000
_E = 320000
_D = 128
_H = 128

_NW = 32
_NBINS = 10240
_PER_TILE = _NBINS // 16

_CHUNK = 10240
_TAIL = _E - 31 * _CHUNK


@functools.cache
def _make_sc_histogram():
    mesh = plsc.VectorSubcoreMesh(core_axis_name="c", subcore_axis_name="s")
    return functools.partial(
        pl.kernel,
        out_type=(jax.ShapeDtypeStruct((1, _NBINS), jnp.float32),
                  jax.ShapeDtypeStruct((1, _NBINS), jnp.float32)),
        mesh=mesh,
        scratch_types=[
            pltpu.VMEM((2, _CHUNK), jnp.int32),
            pltpu.VMEM((_CHUNK,), jnp.int32),
            pltpu.VMEM((_CHUNK,), jnp.float32),
            pltpu.VMEM((_PER_TILE,), jnp.float32),
            pltpu.VMEM_SHARED((_NBINS,), jnp.float32),
        ],
    )(_sc_histogram_body)


def _sc_histogram_body(edge_hbm, out0_hbm, out1_hbm, idx_v, idx1_v, ones_v,
                       zeros_v, counts_sh):
    cid = lax.axis_index("c")
    sid = lax.axis_index("s")
    wid = sid * 2 + cid

    one16 = jnp.ones((16,), jnp.float32)
    zero16 = jnp.zeros((16,), jnp.float32)
    for i in range(_CHUNK // 16):
        ones_v[pl.ds(i * 16, 16)] = one16
    for i in range(_PER_TILE // 16):
        zeros_v[pl.ds(i * 16, 16)] = zero16

    pltpu.sync_copy(zeros_v, counts_sh.at[pl.ds(sid * _PER_TILE, _PER_TILE)])
    @pl.when(wid < 31)
    def _():
        pltpu.sync_copy(edge_hbm.at[:, pl.ds(wid * _CHUNK, _CHUNK)], idx_v)

    @pl.when(wid == 31)
    def _():
        pltpu.sync_copy(edge_hbm.at[:, pl.ds(31 * _CHUNK, _TAIL)],
                        idx_v.at[:, pl.ds(0, _TAIL)])

    def compact(j, carry):
        base = j * 64
        idx1_v[pl.ds(base, 16)] = idx_v[0, pl.ds(base, 16)]
        idx1_v[pl.ds(base + 16, 16)] = idx_v[0, pl.ds(base + 16, 16)]
        idx1_v[pl.ds(base + 32, 16)] = idx_v[0, pl.ds(base + 32, 16)]
        idx1_v[pl.ds(base + 48, 16)] = idx_v[0, pl.ds(base + 48, 16)]
        return carry

    lax.fori_loop(0, _CHUNK // 64, compact, 0)
    plsc.subcore_barrier()

    @pl.when(wid < 31)
    def _():
        pltpu.sync_copy(ones_v, counts_sh.at[idx1_v], add=True)

    @pl.when(wid == 31)
    def _():
        pltpu.sync_copy(ones_v.at[pl.ds(0, _TAIL)],
                        counts_sh.at[idx1_v.at[pl.ds(0, _TAIL)]], add=True)

    plsc.subcore_barrier()

    @pl.when(cid == 0)
    def _():
        pltpu.sync_copy(counts_sh.at[pl.ds(sid * _PER_TILE, _PER_TILE)],
                        out0_hbm.at[0, pl.ds(sid * _PER_TILE, _PER_TILE)])

    @pl.when(cid == 1)
    def _():
        pltpu.sync_copy(counts_sh.at[pl.ds(sid * _PER_TILE, _PER_TILE)],
                        out1_hbm.at[0, pl.ds(sid * _PER_TILE, _PER_TILE)])


def _tc_node_head(x_ref, w1_ref, woutt_ref, y_ref):
    hi = jax.lax.Precision.HIGHEST
    xv = x_ref[...].astype(jnp.bfloat16).astype(jnp.float32)
    w1 = w1_ref[...].astype(jnp.bfloat16).astype(jnp.float32)
    w_row = lax.dot_general(woutt_ref[...], w1, (((1,), (1,)), ((), ())),
                            precision=hi,
                            preferred_element_type=jnp.float32)
    y_ref[...] = lax.dot_general(w_row, xv, (((1,), (1,)), ((), ())),
                                 precision=hi,
                                 preferred_element_type=jnp.float32)


def _tc_final(c0_ref, c1_ref, y_ref, b1_ref, wout_ref, bout_ref, o_ref):
    hi = jax.lax.Precision.HIGHEST
    c1 = (c0_ref[...] + c1_ref[...])[:, :_N]
    out = lax.dot_general(c1, y_ref[...], (((1,), (1,)), ((), ())),
                          precision=hi, preferred_element_type=jnp.float32)
    bias = lax.dot_general(b1_ref[...], wout_ref[...],
                           (((1,), (1,)), ((), ())), precision=hi,
                           preferred_element_type=jnp.float32)
    o_ref[...] = out + jnp.float32(_N) * bias + bout_ref[...]


def kernel(x, edge_index, W1, b1, W_out, b_out):
    counts0, counts1 = _make_sc_histogram()(edge_index)

    y = pl.pallas_call(
        _tc_node_head,
        out_shape=jax.ShapeDtypeStruct((1, _N), jnp.float32),
    )(x, W1, W_out.reshape(1, _H))

    out = pl.pallas_call(
        _tc_final,
        out_shape=jax.ShapeDtypeStruct((1, 1), jnp.float32),
    )(counts0, counts1, y, b1.reshape(1, _H), W_out.reshape(1, _H),
      b_out.reshape(1, 1))
    return out


# speedup vs baseline: 1.0516x; 1.0252x over previous
import functools

import jax
import jax.numpy as jnp
from jax import lax
from jax.experimental import pallas as pl
from jax.experimental.pallas import tpu as pltpu
from jax.experimental.pallas import tpu_sc as plsc

_N = 10000
_E = 320000
_D = 128
_H = 128

_NW = 32
_EPW = _E // _NW
_NBINS = 10240
_PER_TILE = _NBINS // 16
_EPW_PAD = 10240


@functools.cache
def _make_sc_histogram():
    mesh = plsc.VectorSubcoreMesh(core_axis_name="c", subcore_axis_name="s")
    return functools.partial(
        pl.kernel,
        out_type=(jax.ShapeDtypeStruct((1, _NBINS), jnp.float32),
                  jax.ShapeDtypeStruct((1, _NBINS), jnp.float32)),
        mesh=mesh,
        scratch_types=[
            pltpu.VMEM((_EPW_PAD,), jnp.int32),
            pltpu.VMEM((_EPW_PAD,), jnp.float32),
            pltpu.VMEM((_PER_TILE,), jnp.float32),
            pltpu.VMEM_SHARED((_NBINS,), jnp.float32),
        ],
    )(_sc_histogram_body)


def _sc_histogram_body(edge_hbm, out0_hbm, out1_hbm, idx_v, ones_v, zeros_v,
                       counts_sh):
    cid = lax.axis_index("c")
    sid = lax.axis_index("s")
    wid = sid * 2 + cid

    one16 = jnp.ones((16,), jnp.float32)
    zero16 = jnp.zeros((16,), jnp.float32)
    for i in range(_EPW_PAD // 16):
        ones_v[pl.ds(i * 16, 16)] = one16
    for i in range(_PER_TILE // 16):
        zeros_v[pl.ds(i * 16, 16)] = zero16
    lanes = lax.iota(jnp.int32, 16)
    for i in range((_EPW_PAD - _EPW) // 16):
        idx_v[pl.ds(_EPW + i * 16, 16)] = _N + i * 16 + lanes

    pltpu.sync_copy(zeros_v, counts_sh.at[pl.ds(sid * _PER_TILE, _PER_TILE)])
    pltpu.sync_copy(edge_hbm.at[pl.ds(wid * _EPW, _EPW)],
                    idx_v.at[pl.ds(0, _EPW)])
    plsc.subcore_barrier()

    pltpu.sync_copy(ones_v, counts_sh.at[idx_v], add=True)
    plsc.subcore_barrier()

    @pl.when(cid == 0)
    def _():
        pltpu.sync_copy(counts_sh.at[pl.ds(sid * _PER_TILE, _PER_TILE)],
                        out0_hbm.at[0, pl.ds(sid * _PER_TILE, _PER_TILE)])

    @pl.when(cid == 1)
    def _():
        pltpu.sync_copy(counts_sh.at[pl.ds(sid * _PER_TILE, _PER_TILE)],
                        out1_hbm.at[0, pl.ds(sid * _PER_TILE, _PER_TILE)])


def _tc_node_head(x_ref, w1_ref, woutt_ref, y_ref):
    hi = jax.lax.Precision.HIGHEST
    xv = x_ref[...].astype(jnp.bfloat16).astype(jnp.float32)
    w1 = w1_ref[...].astype(jnp.bfloat16).astype(jnp.float32)
    w_row = lax.dot_general(woutt_ref[...], w1, (((1,), (1,)), ((), ())),
                            precision=hi,
                            preferred_element_type=jnp.float32)
    y_ref[...] = lax.dot_general(w_row, xv, (((1,), (1,)), ((), ())),
                                 precision=hi,
                                 preferred_element_type=jnp.float32)


def _tc_final(c0_ref, c1_ref, y_ref, b1_ref, wout_ref, bout_ref, o_ref):
    hi = jax.lax.Precision.HIGHEST
    c1 = (c0_ref[...] + c1_ref[...])[:, :_N]
    out = lax.dot_general(c1, y_ref[...], (((1,), (1,)), ((), ())),
                          precision=hi, preferred_element_type=jnp.float32)
    bias = lax.dot_general(b1_ref[...], wout_ref[...],
                           (((1,), (1,)), ((), ())), precision=hi,
                           preferred_element_type=jnp.float32)
    o_ref[...] = out + jnp.float32(_N) * bias + bout_ref[...]


def kernel(x, edge_index, W1, b1, W_out, b_out):
    counts0, counts1 = _make_sc_histogram()(edge_index.reshape(2 * _E))

    y = pl.pallas_call(
        _tc_node_head,
        out_shape=jax.ShapeDtypeStruct((1, _N), jnp.float32),
    )(x, W1, W_out.reshape(1, _H))

    out = pl.pallas_call(
        _tc_final,
        out_shape=jax.ShapeDtypeStruct((1, 1), jnp.float32),
    )(counts0, counts1, y, b1.reshape(1, _H), W_out.reshape(1, _H),
      b_out.reshape(1, 1))
    return out
